# baseline (speedup 1.0000x reference)
import functools

import jax
import jax.numpy as jnp
import numpy as np
from jax import lax
from jax.experimental import pallas as pl
from jax.experimental.pallas import tpu as pltpu
from jax.experimental.pallas import tpu_sc as plsc

NU = 25000
NI = 25000
NN = NU + NI
EE = 1600000
DD = 32
BB = 4096
NC = 2
NS = 16
NW = NC * NS

EPW = EE // NW
SUB = 4
IDX = 100
CH = SUB * IDX
NCHUNK = EPW // CH
NP = 50048
RPT = NP // NS
ZR = 184
NB = BB // NW

_mesh = plsc.VectorSubcoreMesh(core_axis_name="c", subcore_axis_name="s")


@functools.partial(
    pl.kernel,
    out_type=jax.ShapeDtypeStruct((NC * NP, DD), jnp.float32),
    mesh=_mesh,
    scratch_types=[
        pltpu.VMEM_SHARED((NP, DD), jnp.float32),
        [pltpu.VMEM((2 * CH,), jnp.int32)] * 3,
        [pltpu.VMEM((CH,), jnp.float32)] * 3,
        [pltpu.VMEM((CH, DD), jnp.float32)] * 2,
        [pltpu.SemaphoreType.DMA] * 3,
        [pltpu.SemaphoreType.DMA] * 2,
        [pltpu.SemaphoreType.DMA] * 2,
    ],
    compiler_params=pltpu.CompilerParams(use_tc_tiling_on_sc=False),
)
def _edge_pass(comb_hbm, val_hbm, table, zeros_hbm, out, acc, ebuf, valv,
               msgs, sem_lin, sem_gat, sem_sct):
    c = lax.axis_index("c")
    s = lax.axis_index("s")
    w = s * NC + c

    def issue_lin(ci, r):
        gc = w * NCHUNK + ci
        pltpu.async_copy(comb_hbm.at[pl.ds(gc * 2 * CH, 2 * CH)], ebuf[r],
                         sem_lin[r])
        pltpu.async_copy(val_hbm.at[pl.ds(w * EPW + ci * CH, CH)],
                         valv[r], sem_lin[r])

    def wait_lin(r):
        pltpu.make_async_copy(comb_hbm.at[pl.ds(0, 2 * CH)], ebuf[r],
                              sem_lin[r]).wait()
        pltpu.make_async_copy(val_hbm.at[pl.ds(0, CH)], valv[r],
                              sem_lin[r]).wait()

    def issue_gat(b, r):
        pltpu.async_copy(table.at[ebuf[r].at[pl.ds(CH, CH)]], msgs[b],
                         sem_gat[b])

    def wait_gat(b):
        pltpu.make_async_copy(table.at[pl.ds(0, CH)], msgs[b],
                              sem_gat[b]).wait()

    zero16 = lax.iota(jnp.int32, 16) * 0

    def scale(b, r):
        m = msgs[b]
        va = valv[r]

        @plsc.parallel_loop(0, CH // 16, unroll=5)
        def _(g):
            vv = va[pl.ds(g * 16, 16)]
            for e in range(16):
                k = g * 16 + e
                sp = vv.at[zero16 + e].get(mode="promise_in_bounds")
                m[k, 0:16] = m[k, 0:16] * sp
                m[k, 16:32] = m[k, 16:32] * sp

    def issue_sct(b, r):
        pltpu.async_copy(msgs[b], acc.at[ebuf[r].at[pl.ds(0, CH)]],
                         sem_sct[b], add=True)

    def wait_sct(b):
        pltpu.make_async_copy(zeros_hbm.at[pl.ds(0, CH)], msgs[b],
                              sem_sct[b]).wait()

    def steady(ci, b, r, first=False, do_lin=True):
        wait_lin((r + 1) % 3)
        if not first:
            wait_sct(1 - b)
        issue_gat(1 - b, (r + 1) % 3)
        wait_gat(b)
        scale(b, r)
        issue_sct(b, r)
        if do_lin:
            issue_lin(ci + 2, (r + 2) % 3)

    issue_lin(0, 0)
    issue_lin(1, 1)
    pltpu.sync_copy(zeros_hbm, acc.at[pl.ds(s * RPT, RPT)])
    plsc.subcore_barrier()
    wait_lin(0)
    issue_gat(0, 0)

    steady(0, 0, 0, first=True)

    @pl.loop(0, (NCHUNK - 5) // 6)
    def _(p):
        ci = 1 + 6 * p
        for j in range(6):
            steady(ci + j, (1 + j) % 2, (1 + j) % 3)

    steady(NCHUNK - 4, 1, 1)
    steady(NCHUNK - 3, 0, 2)
    steady(NCHUNK - 2, 1, 0, do_lin=False)
    wait_sct(1)
    wait_gat(0)
    scale(0, 1)
    issue_sct(0, 1)
    wait_sct(0)

    plsc.subcore_barrier()
    pltpu.sync_copy(acc.at[pl.ds(s * RPT, RPT)],
                    out.at[pl.ds(c * NP + s * RPT, RPT)])


MRW = NP // NW
MCH = MRW // 4


@functools.partial(
    pl.kernel,
    out_type=jax.ShapeDtypeStruct((NP, DD), jnp.float32),
    mesh=_mesh,
    scratch_types=[
        [pltpu.VMEM((MCH, DD), jnp.float32)] * 2,
        pltpu.SemaphoreType.DMA,
    ],
    compiler_params=pltpu.CompilerParams(use_tc_tiling_on_sc=False),
)
def _merge(part, out, bufs, sem):
    c = lax.axis_index("c")
    s = lax.axis_index("s")
    w = s * NC + c
    a, bb = bufs
    for k in range(4):
        r0 = w * MRW + k * MCH
        cp0 = pltpu.async_copy(part.at[pl.ds(r0, MCH)], a, sem)
        cp1 = pltpu.async_copy(part.at[pl.ds(NP + r0, MCH)], bb, sem)
        cp0.wait()
        cp1.wait()

        @pl.loop(0, MCH, unroll=4)
        def _(i):
            a[i, 0:16] = a[i, 0:16] + bb[i, 0:16]
            a[i, 16:32] = a[i, 16:32] + bb[i, 16:32]

        pltpu.sync_copy(a, out.at[pl.ds(r0, MCH)])


@functools.partial(
    pl.kernel,
    out_type=jax.ShapeDtypeStruct((BB,), jnp.float32),
    mesh=_mesh,
    scratch_types=[
        pltpu.VMEM((NB,), jnp.int32),
        pltpu.VMEM((NB,), jnp.int32),
        pltpu.VMEM((NB, DD), jnp.float32),
        pltpu.VMEM((NB, DD), jnp.float32),
        pltpu.VMEM((NB, DD), jnp.float32),
        pltpu.VMEM((NB,), jnp.float32),
        pltpu.SemaphoreType.DMA,
    ],
    compiler_params=pltpu.CompilerParams(use_tc_tiling_on_sc=False,
                                         needs_layout_passes=False),
)
def _final(users, items, uemb, iemb, t1, t2, part2, gamma, uidx, iidx, usum,
           isum, tmp, gout, sem):
    c = lax.axis_index("c")
    s = lax.axis_index("s")
    w = s * NC + c
    b0 = w * NB
    pltpu.sync_copy(users.at[pl.ds(b0, NB)], uidx)
    pltpu.sync_copy(items.at[pl.ds(b0, NB)], iidx)

    def _shift(idx, off):
        @pl.loop(0, NB // 16)
        def _(i):
            idx[pl.ds(i * 16, 16)] = idx[pl.ds(i * 16, 16)] + off

    def _add_rows(dst, src):
        @pl.loop(0, NB, unroll=4)
        def _(i):
            dst[i, 0:16] = dst[i, 0:16] + src[i, 0:16]
            dst[i, 16:32] = dst[i, 16:32] + src[i, 16:32]

    def _gather_add(tbl, idx, dst):
        pltpu.async_copy(tbl.at[idx], tmp, sem).wait()
        _add_rows(dst, tmp)

    pltpu.async_copy(uemb.at[uidx], usum, sem).wait()
    _gather_add(t1, uidx, usum)
    _gather_add(t2, uidx, usum)
    _gather_add(part2, uidx, usum)
    _shift(uidx, NP)
    _gather_add(part2, uidx, usum)

    pltpu.async_copy(iemb.at[iidx], isum, sem).wait()
    _shift(iidx, NU)
    _gather_add(t1, iidx, isum)
    _gather_add(t2, iidx, isum)
    _gather_add(part2, iidx, isum)
    _shift(iidx, NP)
    _gather_add(part2, iidx, isum)

    lanes = lax.iota(jnp.int32, 16)

    @pl.loop(0, NB // 16)
    def _(g):
        gvec = jnp.zeros((16,), jnp.float32)
        for e in range(16):
            b = g * 16 + e
            prod = (usum[b, 0:16] * isum[b, 0:16]
                    + usum[b, 16:32] * isum[b, 16:32])
            gvec = jnp.where(lanes == e, jnp.sum(prod), gvec)
        gout[pl.ds(g * 16, 16)] = gvec * (1.0 / 16.0)

    pltpu.sync_copy(gout, gamma.at[pl.ds(b0, NB)])


def kernel(users, items, adj_indices, adj_values, user_emb, item_emb):
    row2 = adj_indices[0].reshape(-1, CH)
    col2 = adj_indices[1].reshape(-1, CH)
    comb = jnp.concatenate([row2, col2], axis=1).reshape(-1)
    table0 = jnp.concatenate([user_emb, item_emb], axis=0)
    zeros = jnp.zeros((RPT, DD), jnp.float32)

    part0 = _edge_pass(comb, adj_values, table0, zeros)
    t1 = _merge(part0)
    part1 = _edge_pass(comb, adj_values, t1, zeros)
    t2 = _merge(part1)
    part2 = _edge_pass(comb, adj_values, t2, zeros)

    gamma = _final(users, items, user_emb, item_emb, t1, t2, part2)
    return gamma


# speedup vs baseline: 1.1413x; 1.1413x over previous
import functools

import jax
import jax.numpy as jnp
import numpy as np
from jax import lax
from jax.experimental import pallas as pl
from jax.experimental.pallas import tpu as pltpu
from jax.experimental.pallas import tpu_sc as plsc

NU = 25000
NI = 25000
NN = NU + NI
EE = 1600000
DD = 32
BB = 4096
NC = 2
NS = 16
NW = NC * NS

EPW = EE // NW
SUB = 4
IDX = 100
CH = SUB * IDX
NCHUNK = EPW // CH
NP = 50048
RPT = NP // NS
ZR = 184
NB = BB // NW

_mesh = plsc.VectorSubcoreMesh(core_axis_name="c", subcore_axis_name="s")


@functools.partial(
    pl.kernel,
    out_type=jax.ShapeDtypeStruct((NC * NP, DD), jnp.float32),
    mesh=_mesh,
    scratch_types=[
        pltpu.VMEM_SHARED((NP, DD), jnp.float32),
        [pltpu.VMEM((2 * CH,), jnp.int32)] * 3,
        [pltpu.VMEM((CH,), jnp.float32)] * 3,
        [pltpu.VMEM((CH, DD), jnp.float32)] * 2,
        [pltpu.SemaphoreType.DMA] * 3,
        [pltpu.SemaphoreType.DMA] * 2,
        [pltpu.SemaphoreType.DMA] * 2,
    ],
    compiler_params=pltpu.CompilerParams(use_tc_tiling_on_sc=False),
)
def _edge_pass(adj_hbm, val_hbm, table, zeros_hbm, out, acc, ebuf, valv,
               msgs, sem_lin, sem_gat, sem_sct):
    c = lax.axis_index("c")
    s = lax.axis_index("s")
    w = s * NC + c

    def issue_lin(ci, r):
        e0 = w * EPW + ci * CH
        pltpu.async_copy(adj_hbm.at[pl.ds(e0, CH)],
                         ebuf[r].at[pl.ds(0, CH)], sem_lin[r])
        pltpu.async_copy(adj_hbm.at[pl.ds(EE + e0, CH)],
                         ebuf[r].at[pl.ds(CH, CH)], sem_lin[r])
        pltpu.async_copy(val_hbm.at[pl.ds(e0, CH)], valv[r], sem_lin[r])

    def wait_lin(r):
        pltpu.make_async_copy(adj_hbm.at[pl.ds(0, CH)],
                              ebuf[r].at[pl.ds(0, CH)], sem_lin[r]).wait()
        pltpu.make_async_copy(adj_hbm.at[pl.ds(0, CH)],
                              ebuf[r].at[pl.ds(CH, CH)], sem_lin[r]).wait()
        pltpu.make_async_copy(val_hbm.at[pl.ds(0, CH)], valv[r],
                              sem_lin[r]).wait()

    def issue_gat(b, r):
        pltpu.async_copy(table.at[ebuf[r].at[pl.ds(CH, CH)]], msgs[b],
                         sem_gat[b])

    def wait_gat(b):
        pltpu.make_async_copy(table.at[pl.ds(0, CH)], msgs[b],
                              sem_gat[b]).wait()

    zero16 = lax.iota(jnp.int32, 16) * 0

    def scale(b, r):
        m = msgs[b]
        va = valv[r]

        @plsc.parallel_loop(0, CH // 16, unroll=5)
        def _(g):
            vv = va[pl.ds(g * 16, 16)]
            for e in range(16):
                k = g * 16 + e
                sp = vv.at[zero16 + e].get(mode="promise_in_bounds")
                m[k, 0:16] = m[k, 0:16] * sp
                m[k, 16:32] = m[k, 16:32] * sp

    def issue_sct(b, r):
        pltpu.async_copy(msgs[b], acc.at[ebuf[r].at[pl.ds(0, CH)]],
                         sem_sct[b], add=True)

    def wait_sct(b):
        pltpu.make_async_copy(zeros_hbm.at[pl.ds(0, CH)], msgs[b],
                              sem_sct[b]).wait()

    def steady(ci, b, r, first=False, do_lin=True):
        wait_lin((r + 1) % 3)
        if not first:
            wait_sct(1 - b)
        issue_gat(1 - b, (r + 1) % 3)
        wait_gat(b)
        scale(b, r)
        issue_sct(b, r)
        if do_lin:
            issue_lin(ci + 2, (r + 2) % 3)

    issue_lin(0, 0)
    issue_lin(1, 1)
    pltpu.sync_copy(zeros_hbm, acc.at[pl.ds(s * RPT, RPT)])
    plsc.subcore_barrier()
    wait_lin(0)
    issue_gat(0, 0)

    steady(0, 0, 0, first=True)

    @pl.loop(0, (NCHUNK - 5) // 6)
    def _(p):
        ci = 1 + 6 * p
        for j in range(6):
            steady(ci + j, (1 + j) % 2, (1 + j) % 3)

    steady(NCHUNK - 4, 1, 1)
    steady(NCHUNK - 3, 0, 2)
    steady(NCHUNK - 2, 1, 0, do_lin=False)
    wait_sct(1)
    wait_gat(0)
    scale(0, 1)
    issue_sct(0, 1)
    wait_sct(0)

    plsc.subcore_barrier()
    pltpu.sync_copy(acc.at[pl.ds(s * RPT, RPT)],
                    out.at[pl.ds(c * NP + s * RPT, RPT)])


MRW = NP // NW
MCH = MRW // 4


@functools.partial(
    pl.kernel,
    out_type=jax.ShapeDtypeStruct((NP, DD), jnp.float32),
    mesh=_mesh,
    scratch_types=[
        [pltpu.VMEM((MCH, DD), jnp.float32)] * 2,
        pltpu.SemaphoreType.DMA,
    ],
    compiler_params=pltpu.CompilerParams(use_tc_tiling_on_sc=False),
)
def _merge(part, out, bufs, sem):
    c = lax.axis_index("c")
    s = lax.axis_index("s")
    w = s * NC + c
    a, bb = bufs
    for k in range(4):
        r0 = w * MRW + k * MCH
        cp0 = pltpu.async_copy(part.at[pl.ds(r0, MCH)], a, sem)
        cp1 = pltpu.async_copy(part.at[pl.ds(NP + r0, MCH)], bb, sem)
        cp0.wait()
        cp1.wait()

        @pl.loop(0, MCH, unroll=4)
        def _(i):
            a[i, 0:16] = a[i, 0:16] + bb[i, 0:16]
            a[i, 16:32] = a[i, 16:32] + bb[i, 16:32]

        pltpu.sync_copy(a, out.at[pl.ds(r0, MCH)])


@functools.partial(
    pl.kernel,
    out_type=jax.ShapeDtypeStruct((BB,), jnp.float32),
    mesh=_mesh,
    scratch_types=[
        pltpu.VMEM((NB,), jnp.int32),
        pltpu.VMEM((NB,), jnp.int32),
        pltpu.VMEM((NB, DD), jnp.float32),
        pltpu.VMEM((NB, DD), jnp.float32),
        pltpu.VMEM((NB, DD), jnp.float32),
        pltpu.VMEM((NB,), jnp.float32),
        pltpu.SemaphoreType.DMA,
    ],
    compiler_params=pltpu.CompilerParams(use_tc_tiling_on_sc=False,
                                         needs_layout_passes=False),
)
def _final(users, items, uemb, iemb, t1, t2, part2, gamma, uidx, iidx, usum,
           isum, tmp, gout, sem):
    c = lax.axis_index("c")
    s = lax.axis_index("s")
    w = s * NC + c
    b0 = w * NB
    pltpu.sync_copy(users.at[pl.ds(b0, NB)], uidx)
    pltpu.sync_copy(items.at[pl.ds(b0, NB)], iidx)

    def _shift(idx, off):
        @pl.loop(0, NB // 16)
        def _(i):
            idx[pl.ds(i * 16, 16)] = idx[pl.ds(i * 16, 16)] + off

    def _add_rows(dst, src):
        @pl.loop(0, NB, unroll=4)
        def _(i):
            dst[i, 0:16] = dst[i, 0:16] + src[i, 0:16]
            dst[i, 16:32] = dst[i, 16:32] + src[i, 16:32]

    def _gather_add(tbl, idx, dst):
        pltpu.async_copy(tbl.at[idx], tmp, sem).wait()
        _add_rows(dst, tmp)

    pltpu.async_copy(uemb.at[uidx], usum, sem).wait()
    _gather_add(t1, uidx, usum)
    _gather_add(t2, uidx, usum)
    _gather_add(part2, uidx, usum)
    _shift(uidx, NP)
    _gather_add(part2, uidx, usum)

    pltpu.async_copy(iemb.at[iidx], isum, sem).wait()
    _shift(iidx, NU)
    _gather_add(t1, iidx, isum)
    _gather_add(t2, iidx, isum)
    _gather_add(part2, iidx, isum)
    _shift(iidx, NP)
    _gather_add(part2, iidx, isum)

    lanes = lax.iota(jnp.int32, 16)

    @pl.loop(0, NB // 16)
    def _(g):
        gvec = jnp.zeros((16,), jnp.float32)
        for e in range(16):
            b = g * 16 + e
            prod = (usum[b, 0:16] * isum[b, 0:16]
                    + usum[b, 16:32] * isum[b, 16:32])
            gvec = jnp.where(lanes == e, jnp.sum(prod), gvec)
        gout[pl.ds(g * 16, 16)] = gvec * (1.0 / 16.0)

    pltpu.sync_copy(gout, gamma.at[pl.ds(b0, NB)])


def kernel(users, items, adj_indices, adj_values, user_emb, item_emb):
    adjflat = adj_indices.reshape(2 * EE)
    table0 = jnp.concatenate([user_emb, item_emb], axis=0)
    zeros = jnp.zeros((RPT, DD), jnp.float32)

    part0 = _edge_pass(adjflat, adj_values, table0, zeros)
    t1 = _merge(part0)
    part1 = _edge_pass(adjflat, adj_values, t1, zeros)
    t2 = _merge(part1)
    part2 = _edge_pass(adjflat, adj_values, t2, zeros)

    gamma = _final(users, items, user_emb, item_emb, t1, t2, part2)
    return gamma
